# baseline (speedup 1.0000x reference)
import functools
import jax
import jax.numpy as jnp
from jax import lax
from jax.experimental import pallas as pl
from jax.experimental.pallas import tpu as pltpu
from jax.experimental.pallas import tpu_sc as plsc

N = 10000
E = 160000
H = 256
HH = H // 2
G = 64
BN = 1000
NB = N // BN

NC = 2
NS = 16
EPT = E // NS
CH = 128
NFULL = EPT // CH
TAIL = EPT - NFULL * CH
ACC_ROWS = N
ZPT = ACC_ROWS // NS
HW = HH // 2
OPT = (N // NS) // 8 * 8
OREM = N - OPT * NS


def _bf(x):
    return x.astype(jnp.bfloat16)


def _mlp_body(x_ref, w1_ref, b1_ref, w2_ref, b2_ref, h_ref):
    x = x_ref[...]
    h = jnp.maximum(jnp.dot(_bf(x), _bf(w1_ref[...]), preferred_element_type=jnp.float32) + b1_ref[...], 0.0)
    h = jnp.maximum(jnp.dot(_bf(h), _bf(w2_ref[...]), preferred_element_type=jnp.float32) + b2_ref[...], 0.0)
    h_ref[...] = h


def _mlp_call(x, w1, b1, w2, b2):
    row = pl.BlockSpec((BN, H), lambda i: (i, 0))
    full = pl.BlockSpec((H, H), lambda i: (0, 0))
    bias = pl.BlockSpec((1, H), lambda i: (0, 0))
    return pl.pallas_call(
        _mlp_body,
        grid=(NB,),
        in_specs=[row, full, bias, full, bias],
        out_specs=row,
        out_shape=jax.ShapeDtypeStruct((N, H), jnp.float32),
    )(x, w1, b1, w2, b2)


def _m2_body(h_ref, w_ref, m_ref):
    m_ref[...] = jnp.dot(_bf(h_ref[...]), _bf(w_ref[...]),
                         preferred_element_type=jnp.float32)


def _m2_call(h, w):
    return pl.pallas_call(
        _m2_body,
        grid=(NC, NB),
        in_specs=[pl.BlockSpec((BN, H), lambda j, i: (i, 0)),
                  pl.BlockSpec((H, HH), lambda j, i: (0, j))],
        out_specs=pl.BlockSpec((BN, HH), lambda j, i: (j * NB + i, 0)),
        out_shape=jax.ShapeDtypeStruct((NC * N, HH), jnp.float32),
    )(h, w)


def _gh_body(h_ref, whh_ref, bhh_ref, gh_ref):
    gh_ref[...] = lax.dot_general(_bf(h_ref[...]), _bf(whh_ref[...]),
                                  (((1,), (1,)), ((), ())),
                                  preferred_element_type=jnp.float32) + bhh_ref[...]


def _gh_call(h, whh, bhh):
    return pl.pallas_call(
        _gh_body,
        grid=(NB,),
        in_specs=[pl.BlockSpec((BN, H), lambda i: (i, 0)),
                  pl.BlockSpec((3 * H, H), lambda i: (0, 0)),
                  pl.BlockSpec((1, 3 * H), lambda i: (0, 0))],
        out_specs=pl.BlockSpec((BN, 3 * H), lambda i: (i, 0)),
        out_shape=jax.ShapeDtypeStruct((N, 3 * H), jnp.float32),
    )(h, whh, bhh)


def _gru_body(agga_ref, aggb_ref, gh_ref, h_ref, wih_ref, bih_ref, hn_ref):
    agg = jnp.concatenate([agga_ref[...], aggb_ref[...]], axis=1)
    h = h_ref[...]
    gh = gh_ref[...]
    gi = lax.dot_general(_bf(agg), _bf(wih_ref[...]), (((1,), (1,)), ((), ())),
                         preferred_element_type=jnp.float32) + bih_ref[...]
    r = jax.nn.sigmoid(gi[:, :H] + gh[:, :H])
    z = jax.nn.sigmoid(gi[:, H:2 * H] + gh[:, H:2 * H])
    n = jnp.tanh(gi[:, 2 * H:] + r * gh[:, 2 * H:])
    hn_ref[...] = (1.0 - z) * n + z * h


def _gru_call(agg2, gh, h, wih, bih):
    row = pl.BlockSpec((BN, H), lambda i: (i, 0))
    return pl.pallas_call(
        _gru_body,
        grid=(NB,),
        in_specs=[pl.BlockSpec((BN, HH), lambda i: (i, 0)),
                  pl.BlockSpec((BN, HH), lambda i: (NB + i, 0)),
                  pl.BlockSpec((BN, 3 * H), lambda i: (i, 0)),
                  row,
                  pl.BlockSpec((3 * H, H), lambda i: (0, 0)),
                  pl.BlockSpec((1, 3 * H), lambda i: (0, 0))],
        out_specs=row,
        out_shape=jax.ShapeDtypeStruct((N, H), jnp.float32),
    )(agg2, agg2, gh, h, wih, bih)


def _pool_body(h_ref, batch_ref, ow_ref, ob_ref, out_ref):
    h = h_ref[...]
    b = batch_ref[...]
    onehot = (b == lax.broadcasted_iota(jnp.int32, (1, G), 1)).astype(jnp.float32)
    sums = lax.dot_general(onehot, h, (((0,), (0,)), ((), ())),
                           preferred_element_type=jnp.float32)
    counts = jnp.sum(onehot, axis=0)[:, None]
    pooled = sums / jnp.maximum(counts, 1.0)
    out_ref[...] = jnp.dot(pooled, ow_ref[...], preferred_element_type=jnp.float32) + ob_ref[...]


def _pool_call(h, batch2d, out_w, out_b):
    return pl.pallas_call(
        _pool_body,
        in_specs=[pl.BlockSpec((N, H), lambda: (0, 0)),
                  pl.BlockSpec((N, 1), lambda: (0, 0)),
                  pl.BlockSpec((H, 2), lambda: (0, 0)),
                  pl.BlockSpec((1, 2), lambda: (0, 0))],
        out_specs=pl.BlockSpec((G, 2), lambda: (0, 0)),
        out_shape=jax.ShapeDtypeStruct((G, 2), jnp.float32),
    )(h, batch2d, out_w, out_b)


_MESH = plsc.VectorSubcoreMesh(core_axis_name="c", subcore_axis_name="s")


def _agg_body(m2_hbm, src_hbm, dst_hbm, zeros_hbm, agg2_hbm,
              schunk0, dchunk0, schunk1, dchunk1, dscat0, dscat1,
              schunk_t, dchunk_t, rows0, rows1, acc,
              semg0, semg1, semc0, semc1, semi):
    c_idx = lax.axis_index("c")
    s_idx = lax.axis_index("s")

    pltpu.sync_copy(zeros_hbm, acc.at[pl.ds(s_idx * ZPT, ZPT)])
    plsc.subcore_barrier()

    base = s_idx * EPT
    coff = c_idx * N

    def stage(sc, dc, j):
        pltpu.sync_copy(src_hbm.at[pl.ds(base + j * CH, CH)], sc)
        pltpu.sync_copy(dst_hbm.at[pl.ds(base + j * CH, CH)], dc)
        for k in range(CH // 16):
            sc[pl.ds(k * 16, 16)] = sc[pl.ds(k * 16, 16)] + coff

    def prefetch(sc, dc, j):
        pltpu.async_copy(src_hbm.at[pl.ds(base + j * CH, CH)], sc, semi)
        pltpu.async_copy(dst_hbm.at[pl.ds(base + j * CH, CH)], dc, semi)

    def prefetch_wait(sc, dc, j):
        pltpu.make_async_copy(src_hbm.at[pl.ds(base + j * CH, CH)], sc, semi).wait()
        pltpu.make_async_copy(dst_hbm.at[pl.ds(base + j * CH, CH)], dc, semi).wait()
        for k in range(CH // 16):
            sc[pl.ds(k * 16, 16)] = sc[pl.ds(k * 16, 16)] + coff

    NP = NFULL // 2
    stage(schunk0, dchunk0, 0)
    pltpu.async_copy(m2_hbm.at[schunk0], rows0, semg0)
    stage(schunk1, dchunk1, 1)

    def pair(i, carry):
        pltpu.async_copy(m2_hbm.at[schunk1], rows1, semg1)
        pltpu.make_async_copy(m2_hbm.at[schunk0], rows0, semg0).wait()
        last = i == NP - 1
        for k in range(CH // 16):
            dscat0[pl.ds(k * 16, 16)] = dchunk0[pl.ds(k * 16, 16)]

        @pl.when(~last)
        def _():
            prefetch(schunk0, dchunk0, 2 * i + 2)
        pltpu.sync_copy(rows0, acc.at[dscat0], add=True)

        @pl.when(~last)
        def _():
            prefetch_wait(schunk0, dchunk0, 2 * i + 2)
            pltpu.async_copy(m2_hbm.at[schunk0], rows0, semg0)

        pltpu.make_async_copy(m2_hbm.at[schunk1], rows1, semg1).wait()
        for k in range(CH // 16):
            dscat1[pl.ds(k * 16, 16)] = dchunk1[pl.ds(k * 16, 16)]

        @pl.when(~last)
        def _():
            prefetch(schunk1, dchunk1, 2 * i + 3)
        pltpu.sync_copy(rows1, acc.at[dscat1], add=True)

        @pl.when(~last)
        def _():
            prefetch_wait(schunk1, dchunk1, 2 * i + 3)
        return carry

    lax.fori_loop(0, NP, pair, 0)

    pltpu.sync_copy(src_hbm.at[pl.ds(base + NFULL * CH, TAIL)], schunk_t)
    pltpu.sync_copy(dst_hbm.at[pl.ds(base + NFULL * CH, TAIL)], dchunk_t)
    schunk_t[...] = schunk_t[...] + coff
    pltpu.async_copy(m2_hbm.at[schunk_t], rows0.at[pl.ds(0, TAIL)], semg0).wait()
    pltpu.sync_copy(rows0.at[pl.ds(0, TAIL)], acc.at[dchunk_t], add=True)

    plsc.subcore_barrier()
    obase = s_idx * OPT
    pltpu.sync_copy(acc.at[pl.ds(obase, OPT)],
                    agg2_hbm.at[pl.ds(c_idx * N + obase, OPT)])

    @pl.when(s_idx == NS - 1)
    def _():
        pltpu.sync_copy(acc.at[pl.ds(OPT * NS, OREM)],
                        agg2_hbm.at[pl.ds(c_idx * N + OPT * NS, OREM)])


def _agg_call(m2, src, dst, zeros):
    return pl.kernel(
        _agg_body,
        out_type=jax.ShapeDtypeStruct((NC * N, HH), jnp.float32),
        mesh=_MESH,
        scratch_types=[
            pltpu.VMEM((CH,), jnp.int32),
            pltpu.VMEM((CH,), jnp.int32),
            pltpu.VMEM((CH,), jnp.int32),
            pltpu.VMEM((CH,), jnp.int32),
            pltpu.VMEM((CH,), jnp.int32),
            pltpu.VMEM((CH,), jnp.int32),
            pltpu.VMEM((TAIL,), jnp.int32),
            pltpu.VMEM((TAIL,), jnp.int32),
            pltpu.VMEM((CH, HH), jnp.float32),
            pltpu.VMEM((CH, HH), jnp.float32),
            pltpu.VMEM_SHARED((ACC_ROWS, HH), jnp.float32),
            pltpu.SemaphoreType.DMA,
            pltpu.SemaphoreType.DMA,
            pltpu.SemaphoreType.DMA,
            pltpu.SemaphoreType.DMA,
            pltpu.SemaphoreType.DMA,
        ],
    )(m2, src, dst, zeros)


def kernel(x, edge_index, batch, mlp_w1, mlp_b1, mlp_w2, mlp_b2, ggc_w,
           gru_wih, gru_whh, gru_bih, gru_bhh, out_w, out_b):
    src = edge_index[0].astype(jnp.int32)
    dst = edge_index[1].astype(jnp.int32)
    batch2d = batch.astype(jnp.int32).reshape(N, 1)
    b1 = mlp_b1.reshape(1, H)
    b2 = mlp_b2.reshape(1, H)
    bih = gru_bih.reshape(1, 3 * H)
    bhh = gru_bhh.reshape(1, 3 * H)
    ob = out_b.reshape(1, 2)
    zeros = jnp.zeros((ZPT, HH), jnp.float32)

    h = _mlp_call(x, mlp_w1, b1, mlp_w2, b2)
    for i in range(3):
        m2 = _m2_call(h, ggc_w[i])
        agg2 = _agg_call(m2, src, dst, zeros)
        gh = _gh_call(h, gru_whh, bhh)
        h = _gru_call(agg2, gh, h, gru_wih, bih)
    return _pool_call(h, batch2d, out_w, ob)


# speedup vs baseline: 1.0762x; 1.0762x over previous
import functools
import jax
import jax.numpy as jnp
from jax import lax
from jax.experimental import pallas as pl
from jax.experimental.pallas import tpu as pltpu
from jax.experimental.pallas import tpu_sc as plsc

N = 10000
E = 160000
H = 256
HH = H // 2
G = 64
BN = 1000
NB = N // BN

NC = 2
NS = 16
EPT = E // NS
CH = 128
NFULL = EPT // CH
TAIL = EPT - NFULL * CH
ACC_ROWS = N
ZPT = ACC_ROWS // NS
HW = HH // 2
OPT = (N // NS) // 8 * 8
OREM = N - OPT * NS


def _mlp_body(x_ref, w1_ref, b1_ref, w2_ref, b2_ref, h_ref):
    x = x_ref[...]
    h = jnp.maximum(jnp.dot(x, w1_ref[...], preferred_element_type=jnp.float32) + b1_ref[...], 0.0)
    h = jnp.maximum(jnp.dot(h, w2_ref[...], preferred_element_type=jnp.float32) + b2_ref[...], 0.0)
    h_ref[...] = h


def _mlp_call(x, w1, b1, w2, b2):
    row = pl.BlockSpec((BN, H), lambda i: (i, 0))
    full = pl.BlockSpec((H, H), lambda i: (0, 0))
    bias = pl.BlockSpec((1, H), lambda i: (0, 0))
    return pl.pallas_call(
        _mlp_body,
        grid=(NB,),
        in_specs=[row, full, bias, full, bias],
        out_specs=row,
        out_shape=jax.ShapeDtypeStruct((N, H), jnp.float32),
    )(x, w1, b1, w2, b2)


def _m2_body(h_ref, w_ref, m_ref):
    m_ref[...] = jnp.dot(h_ref[...], w_ref[...], preferred_element_type=jnp.float32)


def _m2_call(h, w):
    return pl.pallas_call(
        _m2_body,
        grid=(NC, NB),
        in_specs=[pl.BlockSpec((BN, H), lambda j, i: (i, 0)),
                  pl.BlockSpec((H, HH), lambda j, i: (0, j))],
        out_specs=pl.BlockSpec((BN, HH), lambda j, i: (j * NB + i, 0)),
        out_shape=jax.ShapeDtypeStruct((NC * N, HH), jnp.float32),
    )(h, w)


def _gru_body(agga_ref, aggb_ref, h_ref, wih_ref, whh_ref, bih_ref, bhh_ref,
              hn_ref):
    agg = jnp.concatenate([agga_ref[...], aggb_ref[...]], axis=1)
    h = h_ref[...]
    gi = lax.dot_general(agg, wih_ref[...], (((1,), (1,)), ((), ())),
                         preferred_element_type=jnp.float32) + bih_ref[...]
    gh = lax.dot_general(h, whh_ref[...], (((1,), (1,)), ((), ())),
                         preferred_element_type=jnp.float32) + bhh_ref[...]
    r = jax.nn.sigmoid(gi[:, :H] + gh[:, :H])
    z = jax.nn.sigmoid(gi[:, H:2 * H] + gh[:, H:2 * H])
    n = jnp.tanh(gi[:, 2 * H:] + r * gh[:, 2 * H:])
    hn_ref[...] = (1.0 - z) * n + z * h


def _gru_call(agg2, h, wih, whh, bih, bhh):
    row = pl.BlockSpec((BN, H), lambda i: (i, 0))
    wspec = pl.BlockSpec((3 * H, H), lambda i: (0, 0))
    bspec = pl.BlockSpec((1, 3 * H), lambda i: (0, 0))
    return pl.pallas_call(
        _gru_body,
        grid=(NB,),
        in_specs=[pl.BlockSpec((BN, HH), lambda i: (i, 0)),
                  pl.BlockSpec((BN, HH), lambda i: (NB + i, 0)),
                  row, wspec, wspec, bspec, bspec],
        out_specs=row,
        out_shape=jax.ShapeDtypeStruct((N, H), jnp.float32),
    )(agg2, agg2, h, wih, whh, bih, bhh)


def _pool_body(h_ref, batch_ref, ow_ref, ob_ref, out_ref):
    h = h_ref[...]
    b = batch_ref[...]
    onehot = (b == lax.broadcasted_iota(jnp.int32, (1, G), 1)).astype(jnp.float32)
    sums = lax.dot_general(onehot, h, (((0,), (0,)), ((), ())),
                           preferred_element_type=jnp.float32)
    counts = jnp.sum(onehot, axis=0)[:, None]
    pooled = sums / jnp.maximum(counts, 1.0)
    out_ref[...] = jnp.dot(pooled, ow_ref[...], preferred_element_type=jnp.float32) + ob_ref[...]


def _pool_call(h, batch2d, out_w, out_b):
    return pl.pallas_call(
        _pool_body,
        in_specs=[pl.BlockSpec((N, H), lambda: (0, 0)),
                  pl.BlockSpec((N, 1), lambda: (0, 0)),
                  pl.BlockSpec((H, 2), lambda: (0, 0)),
                  pl.BlockSpec((1, 2), lambda: (0, 0))],
        out_specs=pl.BlockSpec((G, 2), lambda: (0, 0)),
        out_shape=jax.ShapeDtypeStruct((G, 2), jnp.float32),
    )(h, batch2d, out_w, out_b)


_MESH = plsc.VectorSubcoreMesh(core_axis_name="c", subcore_axis_name="s")


def _agg_body(m2_hbm, src_hbm, dst_hbm, zeros_hbm, agg2_hbm,
              schunk0, dchunk0, schunk1, dchunk1, dscat0, dscat1,
              schunk_t, dchunk_t, rows0, rows1, acc,
              semg0, semg1, semc0, semc1, semi):
    c_idx = lax.axis_index("c")
    s_idx = lax.axis_index("s")

    pltpu.sync_copy(zeros_hbm, acc.at[pl.ds(s_idx * ZPT, ZPT)])
    plsc.subcore_barrier()

    base = s_idx * EPT
    coff = c_idx * N

    def stage(sc, dc, j):
        pltpu.sync_copy(src_hbm.at[pl.ds(base + j * CH, CH)], sc)
        pltpu.sync_copy(dst_hbm.at[pl.ds(base + j * CH, CH)], dc)
        for k in range(CH // 16):
            sc[pl.ds(k * 16, 16)] = sc[pl.ds(k * 16, 16)] + coff

    def prefetch(sc, dc, j):
        pltpu.async_copy(src_hbm.at[pl.ds(base + j * CH, CH)], sc, semi)
        pltpu.async_copy(dst_hbm.at[pl.ds(base + j * CH, CH)], dc, semi)

    def prefetch_wait(sc, dc, j):
        pltpu.make_async_copy(src_hbm.at[pl.ds(base + j * CH, CH)], sc, semi).wait()
        pltpu.make_async_copy(dst_hbm.at[pl.ds(base + j * CH, CH)], dc, semi).wait()
        for k in range(CH // 16):
            sc[pl.ds(k * 16, 16)] = sc[pl.ds(k * 16, 16)] + coff

    NP = NFULL // 2
    stage(schunk0, dchunk0, 0)
    pltpu.async_copy(m2_hbm.at[schunk0], rows0, semg0)
    stage(schunk1, dchunk1, 1)

    def pair(i, carry):
        pltpu.async_copy(m2_hbm.at[schunk1], rows1, semg1)
        pltpu.make_async_copy(m2_hbm.at[schunk0], rows0, semg0).wait()
        last = i == NP - 1
        for k in range(CH // 16):
            dscat0[pl.ds(k * 16, 16)] = dchunk0[pl.ds(k * 16, 16)]

        @pl.when(~last)
        def _():
            prefetch(schunk0, dchunk0, 2 * i + 2)
        pltpu.sync_copy(rows0, acc.at[dscat0], add=True)

        @pl.when(~last)
        def _():
            prefetch_wait(schunk0, dchunk0, 2 * i + 2)
            pltpu.async_copy(m2_hbm.at[schunk0], rows0, semg0)

        pltpu.make_async_copy(m2_hbm.at[schunk1], rows1, semg1).wait()
        for k in range(CH // 16):
            dscat1[pl.ds(k * 16, 16)] = dchunk1[pl.ds(k * 16, 16)]

        @pl.when(~last)
        def _():
            prefetch(schunk1, dchunk1, 2 * i + 3)
        pltpu.sync_copy(rows1, acc.at[dscat1], add=True)

        @pl.when(~last)
        def _():
            prefetch_wait(schunk1, dchunk1, 2 * i + 3)
        return carry

    lax.fori_loop(0, NP, pair, 0)

    pltpu.sync_copy(src_hbm.at[pl.ds(base + NFULL * CH, TAIL)], schunk_t)
    pltpu.sync_copy(dst_hbm.at[pl.ds(base + NFULL * CH, TAIL)], dchunk_t)
    schunk_t[...] = schunk_t[...] + coff
    pltpu.async_copy(m2_hbm.at[schunk_t], rows0.at[pl.ds(0, TAIL)], semg0).wait()
    pltpu.sync_copy(rows0.at[pl.ds(0, TAIL)], acc.at[dchunk_t], add=True)

    plsc.subcore_barrier()
    obase = s_idx * OPT
    pltpu.sync_copy(acc.at[pl.ds(obase, OPT)],
                    agg2_hbm.at[pl.ds(c_idx * N + obase, OPT)])

    @pl.when(s_idx == NS - 1)
    def _():
        pltpu.sync_copy(acc.at[pl.ds(OPT * NS, OREM)],
                        agg2_hbm.at[pl.ds(c_idx * N + OPT * NS, OREM)])


def _agg_call(m2, src, dst, zeros):
    return pl.kernel(
        _agg_body,
        out_type=jax.ShapeDtypeStruct((NC * N, HH), jnp.float32),
        mesh=_MESH,
        scratch_types=[
            pltpu.VMEM((CH,), jnp.int32),
            pltpu.VMEM((CH,), jnp.int32),
            pltpu.VMEM((CH,), jnp.int32),
            pltpu.VMEM((CH,), jnp.int32),
            pltpu.VMEM((CH,), jnp.int32),
            pltpu.VMEM((CH,), jnp.int32),
            pltpu.VMEM((TAIL,), jnp.int32),
            pltpu.VMEM((TAIL,), jnp.int32),
            pltpu.VMEM((CH, HH), jnp.float32),
            pltpu.VMEM((CH, HH), jnp.float32),
            pltpu.VMEM_SHARED((ACC_ROWS, HH), jnp.float32),
            pltpu.SemaphoreType.DMA,
            pltpu.SemaphoreType.DMA,
            pltpu.SemaphoreType.DMA,
            pltpu.SemaphoreType.DMA,
            pltpu.SemaphoreType.DMA,
        ],
    )(m2, src, dst, zeros)


def kernel(x, edge_index, batch, mlp_w1, mlp_b1, mlp_w2, mlp_b2, ggc_w,
           gru_wih, gru_whh, gru_bih, gru_bhh, out_w, out_b):
    src = edge_index[0].astype(jnp.int32)
    dst = edge_index[1].astype(jnp.int32)
    batch2d = batch.astype(jnp.int32).reshape(N, 1)
    b1 = mlp_b1.reshape(1, H)
    b2 = mlp_b2.reshape(1, H)
    bih = gru_bih.reshape(1, 3 * H)
    bhh = gru_bhh.reshape(1, 3 * H)
    ob = out_b.reshape(1, 2)
    zeros = jnp.zeros((ZPT, HH), jnp.float32)

    h = _mlp_call(x, mlp_w1, b1, mlp_w2, b2)
    for i in range(3):
        m2 = _m2_call(h, ggc_w[i])
        agg2 = _agg_call(m2, src, dst, zeros)
        h = _gru_call(agg2, h, gru_wih, gru_whh, bih, bhh)
    return _pool_call(h, batch2d, out_w, ob)


# speedup vs baseline: 1.1282x; 1.0484x over previous
import functools
import jax
import jax.numpy as jnp
from jax import lax
from jax.experimental import pallas as pl
from jax.experimental.pallas import tpu as pltpu
from jax.experimental.pallas import tpu_sc as plsc

N = 10000
E = 160000
H = 256
HH = H // 2
G = 64
BN = 1000
NB = N // BN

NC = 2
NS = 16
EPT = E // NS
CH = 128
NFULL = EPT // CH
TAIL = EPT - NFULL * CH
ACC_ROWS = N
ZPT = ACC_ROWS // NS
HW = HH // 2
OPT = (N // NS) // 8 * 8
OREM = N - OPT * NS


def _mlp_body(x_ref, w1_ref, b1_ref, w2_ref, b2_ref, w0_ref,
              h_ref, m2a_ref, m2b_ref):
    x = x_ref[...]
    h = jnp.maximum(jnp.dot(x, w1_ref[...], preferred_element_type=jnp.float32) + b1_ref[...], 0.0)
    h = jnp.maximum(jnp.dot(h, w2_ref[...], preferred_element_type=jnp.float32) + b2_ref[...], 0.0)
    h_ref[...] = h
    m2 = jnp.dot(h, w0_ref[...], preferred_element_type=jnp.float32)
    m2a_ref[...] = m2[:, :HH]
    m2b_ref[...] = m2[:, HH:]


def _mlp_call(x, w1, b1, w2, b2, w0):
    row = pl.BlockSpec((BN, H), lambda i: (i, 0))
    half = pl.BlockSpec((BN, HH), lambda i: (i, 0))
    full = pl.BlockSpec((H, H), lambda i: (0, 0))
    bias = pl.BlockSpec((1, H), lambda i: (0, 0))
    return pl.pallas_call(
        _mlp_body,
        grid=(NB,),
        in_specs=[row, full, bias, full, bias, full],
        out_specs=[row, half, half],
        out_shape=[jax.ShapeDtypeStruct((N, H), jnp.float32),
                   jax.ShapeDtypeStruct((N, HH), jnp.float32),
                   jax.ShapeDtypeStruct((N, HH), jnp.float32)],
    )(x, w1, b1, w2, b2, w0)


def _gru_gates(agga_ref, aggb_ref, h_ref, wih_ref, whh_ref, bih_ref, bhh_ref):
    agg = jnp.concatenate([agga_ref[...], aggb_ref[...]], axis=1)
    h = h_ref[...]
    gi = lax.dot_general(agg, wih_ref[...], (((1,), (1,)), ((), ())),
                         preferred_element_type=jnp.float32) + bih_ref[...]
    gh = lax.dot_general(h, whh_ref[...], (((1,), (1,)), ((), ())),
                         preferred_element_type=jnp.float32) + bhh_ref[...]
    r = jax.nn.sigmoid(gi[:, :H] + gh[:, :H])
    z = jax.nn.sigmoid(gi[:, H:2 * H] + gh[:, H:2 * H])
    n = jnp.tanh(gi[:, 2 * H:] + r * gh[:, 2 * H:])
    return (1.0 - z) * n + z * h


def _gru_m2_body(agga_ref, aggb_ref, h_ref, wih_ref, whh_ref, bih_ref,
                 bhh_ref, wn_ref, hn_ref, m2a_ref, m2b_ref):
    hn = _gru_gates(agga_ref, aggb_ref, h_ref, wih_ref, whh_ref, bih_ref, bhh_ref)
    hn_ref[...] = hn
    m2 = jnp.dot(hn, wn_ref[...], preferred_element_type=jnp.float32)
    m2a_ref[...] = m2[:, :HH]
    m2b_ref[...] = m2[:, HH:]


def _gru_last_body(agga_ref, aggb_ref, h_ref, wih_ref, whh_ref, bih_ref,
                   bhh_ref, hn_ref):
    hn_ref[...] = _gru_gates(agga_ref, aggb_ref, h_ref, wih_ref, whh_ref,
                             bih_ref, bhh_ref)


def _gru_specs():
    row = pl.BlockSpec((BN, H), lambda i: (i, 0))
    wspec = pl.BlockSpec((3 * H, H), lambda i: (0, 0))
    bspec = pl.BlockSpec((1, 3 * H), lambda i: (0, 0))
    return [pl.BlockSpec((BN, HH), lambda i: (i, 0)),
            pl.BlockSpec((BN, HH), lambda i: (NB + i, 0)),
            row, wspec, wspec, bspec, bspec]


def _gru_m2_call(agg2, h, wih, whh, bih, bhh, wn):
    row = pl.BlockSpec((BN, H), lambda i: (i, 0))
    half = pl.BlockSpec((BN, HH), lambda i: (i, 0))
    full = pl.BlockSpec((H, H), lambda i: (0, 0))
    return pl.pallas_call(
        _gru_m2_body,
        grid=(NB,),
        in_specs=_gru_specs() + [full],
        out_specs=[row, half, half],
        out_shape=[jax.ShapeDtypeStruct((N, H), jnp.float32),
                   jax.ShapeDtypeStruct((N, HH), jnp.float32),
                   jax.ShapeDtypeStruct((N, HH), jnp.float32)],
    )(agg2, agg2, h, wih, whh, bih, bhh, wn)


def _gru_last_call(agg2, h, wih, whh, bih, bhh):
    row = pl.BlockSpec((BN, H), lambda i: (i, 0))
    return pl.pallas_call(
        _gru_last_body,
        grid=(NB,),
        in_specs=_gru_specs(),
        out_specs=row,
        out_shape=jax.ShapeDtypeStruct((N, H), jnp.float32),
    )(agg2, agg2, h, wih, whh, bih, bhh)


def _pool_body(h_ref, batch_ref, ow_ref, ob_ref, out_ref):
    h = h_ref[...]
    b = batch_ref[...]
    onehot = (b == lax.broadcasted_iota(jnp.int32, (1, G), 1)).astype(jnp.float32)
    sums = lax.dot_general(onehot, h, (((0,), (0,)), ((), ())),
                           preferred_element_type=jnp.float32)
    counts = jnp.sum(onehot, axis=0)[:, None]
    pooled = sums / jnp.maximum(counts, 1.0)
    out_ref[...] = jnp.dot(pooled, ow_ref[...], preferred_element_type=jnp.float32) + ob_ref[...]


def _pool_call(h, batch2d, out_w, out_b):
    return pl.pallas_call(
        _pool_body,
        in_specs=[pl.BlockSpec((N, H), lambda: (0, 0)),
                  pl.BlockSpec((N, 1), lambda: (0, 0)),
                  pl.BlockSpec((H, 2), lambda: (0, 0)),
                  pl.BlockSpec((1, 2), lambda: (0, 0))],
        out_specs=pl.BlockSpec((G, 2), lambda: (0, 0)),
        out_shape=jax.ShapeDtypeStruct((G, 2), jnp.float32),
    )(h, batch2d, out_w, out_b)


_MESH = plsc.VectorSubcoreMesh(core_axis_name="c", subcore_axis_name="s")


def _agg_body(m2_hbm, src_hbm, dst_hbm, zeros_hbm, agg2_hbm,
              schunk0, dchunk0, schunk1, dchunk1, dscat0, dscat1,
              schunk_t, dchunk_t, rows0, rows1, acc,
              semg0, semg1, semc0, semc1, semi):
    c_idx = lax.axis_index("c")
    s_idx = lax.axis_index("s")

    pltpu.sync_copy(zeros_hbm, acc.at[pl.ds(s_idx * ZPT, ZPT)])
    plsc.subcore_barrier()

    base = s_idx * EPT
    coff = c_idx * N

    def stage(sc, dc, j):
        pltpu.sync_copy(src_hbm.at[pl.ds(base + j * CH, CH)], sc)
        pltpu.sync_copy(dst_hbm.at[pl.ds(base + j * CH, CH)], dc)
        for k in range(CH // 16):
            sc[pl.ds(k * 16, 16)] = sc[pl.ds(k * 16, 16)] + coff

    def prefetch(sc, dc, j):
        pltpu.async_copy(src_hbm.at[pl.ds(base + j * CH, CH)], sc, semi)
        pltpu.async_copy(dst_hbm.at[pl.ds(base + j * CH, CH)], dc, semi)

    def prefetch_wait(sc, dc, j):
        pltpu.make_async_copy(src_hbm.at[pl.ds(base + j * CH, CH)], sc, semi).wait()
        pltpu.make_async_copy(dst_hbm.at[pl.ds(base + j * CH, CH)], dc, semi).wait()
        for k in range(CH // 16):
            sc[pl.ds(k * 16, 16)] = sc[pl.ds(k * 16, 16)] + coff

    NP = NFULL // 2
    stage(schunk0, dchunk0, 0)
    pltpu.async_copy(m2_hbm.at[schunk0], rows0, semg0)
    stage(schunk1, dchunk1, 1)

    def pair(i, carry):
        pltpu.async_copy(m2_hbm.at[schunk1], rows1, semg1)
        pltpu.make_async_copy(m2_hbm.at[schunk0], rows0, semg0).wait()
        last = i == NP - 1
        for k in range(CH // 16):
            dscat0[pl.ds(k * 16, 16)] = dchunk0[pl.ds(k * 16, 16)]

        @pl.when(~last)
        def _():
            prefetch(schunk0, dchunk0, 2 * i + 2)
        pltpu.sync_copy(rows0, acc.at[dscat0], add=True)

        @pl.when(~last)
        def _():
            prefetch_wait(schunk0, dchunk0, 2 * i + 2)
            pltpu.async_copy(m2_hbm.at[schunk0], rows0, semg0)

        pltpu.make_async_copy(m2_hbm.at[schunk1], rows1, semg1).wait()
        for k in range(CH // 16):
            dscat1[pl.ds(k * 16, 16)] = dchunk1[pl.ds(k * 16, 16)]

        @pl.when(~last)
        def _():
            prefetch(schunk1, dchunk1, 2 * i + 3)
        pltpu.sync_copy(rows1, acc.at[dscat1], add=True)

        @pl.when(~last)
        def _():
            prefetch_wait(schunk1, dchunk1, 2 * i + 3)
        return carry

    lax.fori_loop(0, NP, pair, 0)

    pltpu.sync_copy(src_hbm.at[pl.ds(base + NFULL * CH, TAIL)], schunk_t)
    pltpu.sync_copy(dst_hbm.at[pl.ds(base + NFULL * CH, TAIL)], dchunk_t)
    schunk_t[...] = schunk_t[...] + coff
    pltpu.async_copy(m2_hbm.at[schunk_t], rows0.at[pl.ds(0, TAIL)], semg0).wait()
    pltpu.sync_copy(rows0.at[pl.ds(0, TAIL)], acc.at[dchunk_t], add=True)

    plsc.subcore_barrier()
    obase = s_idx * OPT
    pltpu.sync_copy(acc.at[pl.ds(obase, OPT)],
                    agg2_hbm.at[pl.ds(c_idx * N + obase, OPT)])

    @pl.when(s_idx == NS - 1)
    def _():
        pltpu.sync_copy(acc.at[pl.ds(OPT * NS, OREM)],
                        agg2_hbm.at[pl.ds(c_idx * N + OPT * NS, OREM)])


def _agg_call(m2, src, dst, zeros):
    return pl.kernel(
        _agg_body,
        out_type=jax.ShapeDtypeStruct((NC * N, HH), jnp.float32),
        mesh=_MESH,
        scratch_types=[
            pltpu.VMEM((CH,), jnp.int32),
            pltpu.VMEM((CH,), jnp.int32),
            pltpu.VMEM((CH,), jnp.int32),
            pltpu.VMEM((CH,), jnp.int32),
            pltpu.VMEM((CH,), jnp.int32),
            pltpu.VMEM((CH,), jnp.int32),
            pltpu.VMEM((TAIL,), jnp.int32),
            pltpu.VMEM((TAIL,), jnp.int32),
            pltpu.VMEM((CH, HH), jnp.float32),
            pltpu.VMEM((CH, HH), jnp.float32),
            pltpu.VMEM_SHARED((ACC_ROWS, HH), jnp.float32),
            pltpu.SemaphoreType.DMA,
            pltpu.SemaphoreType.DMA,
            pltpu.SemaphoreType.DMA,
            pltpu.SemaphoreType.DMA,
            pltpu.SemaphoreType.DMA,
        ],
    )(m2, src, dst, zeros)


def kernel(x, edge_index, batch, mlp_w1, mlp_b1, mlp_w2, mlp_b2, ggc_w,
           gru_wih, gru_whh, gru_bih, gru_bhh, out_w, out_b):
    src = edge_index[0].astype(jnp.int32)
    dst = edge_index[1].astype(jnp.int32)
    batch2d = batch.astype(jnp.int32).reshape(N, 1)
    b1 = mlp_b1.reshape(1, H)
    b2 = mlp_b2.reshape(1, H)
    bih = gru_bih.reshape(1, 3 * H)
    bhh = gru_bhh.reshape(1, 3 * H)
    ob = out_b.reshape(1, 2)
    zeros = jnp.zeros((ZPT, HH), jnp.float32)

    h, m2a, m2b = _mlp_call(x, mlp_w1, b1, mlp_w2, b2, ggc_w[0])
    for i in range(3):
        m2 = jnp.concatenate([m2a, m2b], axis=0)
        agg2 = _agg_call(m2, src, dst, zeros)
        if i < 2:
            h, m2a, m2b = _gru_m2_call(agg2, h, gru_wih, gru_whh, bih, bhh,
                                       ggc_w[i + 1])
        else:
            h = _gru_last_call(agg2, h, gru_wih, gru_whh, bih, bhh)
    return _pool_call(h, batch2d, out_w, ob)
